# baseline (speedup 1.0000x reference)
import jax
import jax.numpy as jnp
from jax import lax
from jax.experimental import pallas as pl
from jax.experimental.pallas import tpu as pltpu
from jax.experimental.pallas import tpu_sc as plsc

_B, _C, _H, _W = 16, 20, 128, 128
_N, _DIN, _D = 512, 256, 32
_HW = _H * _W
_CO = _C + _D
_TOTAL = _B * _CO * _HW
_SPAT = _C * _HW
_PLANE = _D * _HW
_NT = 16
_SLAB = _PLANE // _NT
_SPT = _SPAT // _NT
_WPB = _N * _D
_WPT = _WPB // _NT
_NSTR = _WPT // 128
_ZB = 2048
_BPC = _B // 2


def _tc_body(emb_ref, lh_ref, lw_ref, w_ref, b_ref,
             out_ref, data_ref, idx_ref):
    out_ref[...] = jnp.zeros((1, 1, 8, _W), jnp.float32)
    proj = jnp.dot(emb_ref[0], w_ref[...], preferred_element_type=jnp.float32)
    proj = jnp.maximum(proj + b_ref[0, 0][None, :], 0.0)
    lh = jnp.clip(lh_ref[0, 0], 0, _H - 1)
    lw = jnp.clip(lw_ref[0, 0], 0, _W - 1)
    p = lh * _W + lw
    same = p[:, None] == p[None, :]
    col = lax.broadcasted_iota(jnp.int32, (_N, _N), 1)
    row = lax.broadcasted_iota(jnp.int32, (_N, _N), 0)
    has_later = jnp.any(same & (col > row), axis=1)
    sel = jnp.where(same & ~has_later[None, :], 1.0, 0.0)
    data_ref[0] = jnp.dot(sel, proj, preferred_element_type=jnp.float32)
    idx_ref[0] = p[:, None] + \
        lax.broadcasted_iota(jnp.int32, (_N, _D), 1) * _HW


def _sc_body(data_hbm, idx_hbm, spat_hbm, out_hbm,
             plane_a, plane_b, idx_v, data_v, zbig, sem_a, sem_b, sem_sp):
    c = lax.axis_index("c")
    t = lax.axis_index("s")
    sems = (sem_a, sem_b)
    planes = (plane_a, plane_b)

    @pl.loop(0, _ZB // 16)
    def _zfill(i):
        zbig[pl.ds(i * 16, 16)] = jnp.zeros((16,), jnp.float32)

    for h in range(2):
        plane_h = planes[h]

        @pl.loop(0, _SLAB // _ZB)
        def _zslab(i):
            pltpu.sync_copy(zbig, plane_h.at[pl.ds(t * _SLAB + i * _ZB, _ZB)])

    plsc.subcore_barrier()

    @pl.loop(0, _BPC // 2)
    def _pair(g):
        for h in range(2):
            b = c * _BPC + g * 2 + h
            sem_h = sems[h]
            plane_h = planes[h]
            idx_h = idx_v.at[h]

            @pl.when(g > 0)
            def _recycle():
                pltpu.make_async_copy(
                    out_hbm.at[pl.ds(0, _SLAB)],
                    plane_h.at[pl.ds(t * _SLAB, _SLAB)], sem_h).wait()
                plsc.subcore_barrier()
                for j in range(_NSTR):
                    pltpu.sync_copy(zbig.at[pl.ds(0, 128)],
                                    plane_h.at[idx_h.at[j]])
                plsc.subcore_barrier()

            pltpu.async_copy(
                spat_hbm.at[pl.ds(b * _SPAT + t * _SPT, _SPT)],
                out_hbm.at[pl.ds(b * _CO * _HW + t * _SPT, _SPT)], sem_sp)

            pltpu.sync_copy(idx_hbm.at[b * _NT + t], idx_h)
            pltpu.sync_copy(data_hbm.at[b * _NT + t], data_v)
            for j in range(_NSTR):
                pltpu.sync_copy(data_v.at[j], plane_h.at[idx_h.at[j]])
            plsc.subcore_barrier()
            pltpu.async_copy(
                plane_h.at[pl.ds(t * _SLAB, _SLAB)],
                out_hbm.at[pl.ds((b * _CO + _C) * _HW + t * _SLAB, _SLAB)],
                sem_h)

    for h in range(2):
        pltpu.make_async_copy(out_hbm.at[pl.ds(0, _SLAB)],
                              planes[h].at[pl.ds(t * _SLAB, _SLAB)],
                              sems[h]).wait()

    @pl.loop(0, _BPC)
    def _drain_sp(i):
        pltpu.make_async_copy(out_hbm.at[pl.ds(0, _SPT)],
                              plane_a.at[pl.ds(t * _SLAB, _SPT)],
                              sem_sp).wait()


def _make_sc_scatter():
    return pl.kernel(
        _sc_body,
        out_type=(),
        mesh=plsc.VectorSubcoreMesh(core_axis_name="c", subcore_axis_name="s"),
        scratch_types=[
            pltpu.VMEM_SHARED((_PLANE,), jnp.float32),
            pltpu.VMEM_SHARED((_PLANE,), jnp.float32),
            pltpu.VMEM((2, _NSTR, 128), jnp.int32),
            pltpu.VMEM((_NSTR, 128), jnp.float32),
            pltpu.VMEM((_ZB,), jnp.float32),
            pltpu.SemaphoreType.DMA,
            pltpu.SemaphoreType.DMA,
            pltpu.SemaphoreType.DMA,
        ],
    )


def kernel(spatial_info, entity_embeddings, locations, W_proj, b_proj):
    lh = locations[..., 0].reshape(_B, 1, _N)
    lw = locations[..., 1].reshape(_B, 1, _N)
    b3 = b_proj.reshape(1, 1, _D)
    out0, data, idxw = pl.pallas_call(
        _tc_body,
        grid=(_B,),
        in_specs=[
            pl.BlockSpec((1, _N, _DIN), lambda b: (b, 0, 0)),
            pl.BlockSpec((1, 1, _N), lambda b: (b, 0, 0)),
            pl.BlockSpec((1, 1, _N), lambda b: (b, 0, 0)),
            pl.BlockSpec((_DIN, _D), lambda b: (0, 0)),
            pl.BlockSpec((1, 1, _D), lambda b: (0, 0, 0)),
        ],
        out_specs=[
            pl.BlockSpec((1, 1, 8, _W), lambda b: (0, 0, 0, 0)),
            pl.BlockSpec((1, _N, _D), lambda b: (b, 0, 0)),
            pl.BlockSpec((1, _N, _D), lambda b: (b, 0, 0)),
        ],
        out_shape=[
            jax.ShapeDtypeStruct((_B, _CO, _H, _W), jnp.float32),
            jax.ShapeDtypeStruct((_B, _N, _D), jnp.float32),
            jax.ShapeDtypeStruct((_B, _N, _D), jnp.int32),
        ],
    )(entity_embeddings, lh, lw, W_proj, b3)
    data_t = data.reshape(_B * _NT, _NSTR, 128)
    idx_t = idxw.reshape(_B * _NT, _NSTR, 128)
    spat_flat = spatial_info.reshape(_B * _SPAT)
    out_ref = jax.new_ref(out0.reshape(_TOTAL))
    _make_sc_scatter()(data_t, idx_t, spat_flat, out_ref)
    return jax.freeze(out_ref).reshape(_B, _CO, _H, _W)


# speedup vs baseline: 7.2639x; 7.2639x over previous
import jax
import jax.numpy as jnp
from jax import lax
from jax.experimental import pallas as pl
from jax.experimental.pallas import tpu as pltpu
from jax.experimental.pallas import tpu_sc as plsc

_B, _C, _H, _W = 16, 20, 128, 128
_N, _DIN, _D = 512, 256, 32
_HW = _H * _W
_CO = _C + _D
_TOTAL = _B * _CO * _HW
_SPAT = _C * _HW
_PLANE = _D * _HW
_NT = 16
_SLAB = _PLANE // _NT
_SPT = _SPAT // _NT
_WPB = _N * _D
_WPT = _WPB // _NT
_NSTR = _WPT // 128
_ZB = 2048
_BPC = _B // 2


def _tc_body(emb_ref, lh_ref, lw_ref, w_ref, b_ref,
             out_ref, data_ref, idx_ref):
    out_ref[...] = jnp.zeros((1, 1, 8, _W), jnp.float32)
    proj = jnp.dot(emb_ref[0], w_ref[...], preferred_element_type=jnp.float32)
    proj = jnp.maximum(proj + b_ref[0, 0][None, :], 0.0)
    lh = jnp.clip(lh_ref[0, 0], 0, _H - 1)
    lw = jnp.clip(lw_ref[0, 0], 0, _W - 1)
    p = lh * _W + lw
    same = p[:, None] == p[None, :]
    col = lax.broadcasted_iota(jnp.int32, (_N, _N), 1)
    row = lax.broadcasted_iota(jnp.int32, (_N, _N), 0)
    has_later = jnp.any(same & (col > row), axis=1)
    sel = jnp.where(same & ~has_later[None, :], 1.0, 0.0)
    data_ref[0] = jnp.dot(sel, proj, preferred_element_type=jnp.float32)
    idx_ref[0] = p[:, None] + \
        lax.broadcasted_iota(jnp.int32, (_N, _D), 1) * _HW


def _sc_body(data_hbm, idx_hbm, spat_hbm, out_hbm,
             plane_a, plane_b, idx_v, data_v, sp_v, zbig,
             sem_a, sem_b, sem_si_a, sem_si_b, sem_so_a, sem_so_b):
    c = lax.axis_index("c")
    t = lax.axis_index("s")
    sems = (sem_a, sem_b)
    sems_si = (sem_si_a, sem_si_b)
    sems_so = (sem_so_a, sem_so_b)
    planes = (plane_a, plane_b)

    @pl.loop(0, _ZB // 16)
    def _zfill(i):
        zbig[pl.ds(i * 16, 16)] = jnp.zeros((16,), jnp.float32)

    for h in range(2):
        plane_h = planes[h]

        @pl.loop(0, _SLAB // _ZB)
        def _zslab(i):
            pltpu.sync_copy(zbig, plane_h.at[pl.ds(t * _SLAB + i * _ZB, _ZB)])

    plsc.subcore_barrier()

    @pl.loop(0, _BPC // 2)
    def _pair(g):
        for h in range(2):
            b = c * _BPC + g * 2 + h
            sem_h = sems[h]
            plane_h = planes[h]
            idx_h = idx_v.at[h]

            @pl.when(g > 0)
            def _recycle():
                pltpu.make_async_copy(spat_hbm.at[pl.ds(0, _SPT)],
                                      sp_v.at[h], sems_so[h]).wait()
                pltpu.make_async_copy(
                    out_hbm.at[pl.ds(0, _SLAB)],
                    plane_h.at[pl.ds(t * _SLAB, _SLAB)], sem_h).wait()
                plsc.subcore_barrier()
                for j in range(_NSTR):
                    pltpu.sync_copy(zbig.at[pl.ds(0, 128)],
                                    plane_h.at[idx_h.at[j]])
                plsc.subcore_barrier()

            pltpu.async_copy(
                spat_hbm.at[pl.ds(b * _SPAT + t * _SPT, _SPT)],
                sp_v.at[h], sems_si[h])

            pltpu.sync_copy(idx_hbm.at[b * _NT + t], idx_h)
            pltpu.sync_copy(data_hbm.at[b * _NT + t], data_v)
            for j in range(_NSTR):
                pltpu.sync_copy(data_v.at[j], plane_h.at[idx_h.at[j]])
            plsc.subcore_barrier()
            pltpu.async_copy(
                plane_h.at[pl.ds(t * _SLAB, _SLAB)],
                out_hbm.at[pl.ds((b * _CO + _C) * _HW + t * _SLAB, _SLAB)],
                sem_h)
            pltpu.make_async_copy(spat_hbm.at[pl.ds(0, _SPT)],
                                  sp_v.at[h], sems_si[h]).wait()
            pltpu.async_copy(
                sp_v.at[h],
                out_hbm.at[pl.ds(b * _CO * _HW + t * _SPT, _SPT)],
                sems_so[h])

    for h in range(2):
        pltpu.make_async_copy(spat_hbm.at[pl.ds(0, _SPT)],
                              sp_v.at[h], sems_so[h]).wait()
        pltpu.make_async_copy(out_hbm.at[pl.ds(0, _SLAB)],
                              planes[h].at[pl.ds(t * _SLAB, _SLAB)],
                              sems[h]).wait()


def _make_sc_scatter():
    return pl.kernel(
        _sc_body,
        out_type=(),
        mesh=plsc.VectorSubcoreMesh(core_axis_name="c", subcore_axis_name="s"),
        scratch_types=[
            pltpu.VMEM_SHARED((_PLANE,), jnp.float32),
            pltpu.VMEM_SHARED((_PLANE,), jnp.float32),
            pltpu.VMEM((2, _NSTR, 128), jnp.int32),
            pltpu.VMEM((_NSTR, 128), jnp.float32),
            pltpu.VMEM((2, _SPT), jnp.float32),
            pltpu.VMEM((_ZB,), jnp.float32),
            pltpu.SemaphoreType.DMA,
            pltpu.SemaphoreType.DMA,
            pltpu.SemaphoreType.DMA,
            pltpu.SemaphoreType.DMA,
            pltpu.SemaphoreType.DMA,
            pltpu.SemaphoreType.DMA,
        ],
    )


def kernel(spatial_info, entity_embeddings, locations, W_proj, b_proj):
    lh = locations[..., 0].reshape(_B, 1, _N)
    lw = locations[..., 1].reshape(_B, 1, _N)
    b3 = b_proj.reshape(1, 1, _D)
    out0, data, idxw = pl.pallas_call(
        _tc_body,
        grid=(_B,),
        in_specs=[
            pl.BlockSpec((1, _N, _DIN), lambda b: (b, 0, 0)),
            pl.BlockSpec((1, 1, _N), lambda b: (b, 0, 0)),
            pl.BlockSpec((1, 1, _N), lambda b: (b, 0, 0)),
            pl.BlockSpec((_DIN, _D), lambda b: (0, 0)),
            pl.BlockSpec((1, 1, _D), lambda b: (0, 0, 0)),
        ],
        out_specs=[
            pl.BlockSpec((1, 1, 8, _W), lambda b: (0, 0, 0, 0)),
            pl.BlockSpec((1, _N, _D), lambda b: (b, 0, 0)),
            pl.BlockSpec((1, _N, _D), lambda b: (b, 0, 0)),
        ],
        out_shape=[
            jax.ShapeDtypeStruct((_B, _CO, _H, _W), jnp.float32),
            jax.ShapeDtypeStruct((_B, _N, _D), jnp.float32),
            jax.ShapeDtypeStruct((_B, _N, _D), jnp.int32),
        ],
    )(entity_embeddings, lh, lw, W_proj, b3)
    data_t = data.reshape(_B * _NT, _NSTR, 128)
    idx_t = idxw.reshape(_B * _NT, _NSTR, 128)
    spat_flat = spatial_info.reshape(_B * _SPAT)
    out_ref = jax.new_ref(out0.reshape(_TOTAL))
    _make_sc_scatter()(data_t, idx_t, spat_flat, out_ref)
    return jax.freeze(out_ref).reshape(_B, _CO, _H, _W)


# speedup vs baseline: 8.0814x; 1.1125x over previous
import jax
import jax.numpy as jnp
from jax import lax
from jax.experimental import pallas as pl
from jax.experimental.pallas import tpu as pltpu
from jax.experimental.pallas import tpu_sc as plsc

_B, _C, _H, _W = 16, 20, 128, 128
_N, _DIN, _D = 512, 256, 32
_HW = _H * _W
_CO = _C + _D
_TOTAL = _B * _CO * _HW
_SPAT = _C * _HW
_PLANE = _D * _HW
_NT = 16
_SLAB = _PLANE // _NT
_SPT = _SPAT // _NT
_WPB = _N * _D
_WPT = _WPB // _NT
_NSTR = _WPT // 128
_ZB = 2048
_BPC = _B // 2


def _tc_body(emb_ref, lh_ref, lw_ref, w_ref, b_ref,
             out_ref, data_ref, idx_ref):
    out_ref[...] = jnp.zeros((1, 1, 8, _W), jnp.float32)
    proj = jnp.dot(emb_ref[0], w_ref[...], preferred_element_type=jnp.float32)
    proj = jnp.maximum(proj + b_ref[0, 0][None, :], 0.0)
    lh = jnp.clip(lh_ref[0, 0], 0, _H - 1)
    lw = jnp.clip(lw_ref[0, 0], 0, _W - 1)
    p = lh * _W + lw
    same = p[:, None] == p[None, :]
    col = lax.broadcasted_iota(jnp.int32, (_N, _N), 1)
    row = lax.broadcasted_iota(jnp.int32, (_N, _N), 0)
    has_later = jnp.any(same & (col > row), axis=1)
    sel = jnp.where(same & ~has_later[None, :], 1.0, 0.0)
    dataT = lax.dot_general(proj, sel, (((0,), (1,)), ((), ())),
                            preferred_element_type=jnp.float32)
    data_ref[...] = dataT.reshape(_NT, 8, 128)
    p8 = jnp.concatenate([p.reshape(4, 128), p.reshape(4, 128)], axis=0)
    dfac = 2 * lax.broadcasted_iota(jnp.int32, (_NT, 8, 128), 0) + \
        lax.broadcasted_iota(jnp.int32, (_NT, 8, 128), 1) // 4
    idx_ref[...] = p8[None] + dfac * _HW


def _sc_body(data_hbm, idx_hbm, spat_hbm, out_hbm,
             plane_a, plane_b, idx_v, data_v, sp_v, zbig,
             sem_a, sem_b, sem_si_a, sem_si_b, sem_so_a, sem_so_b):
    c = lax.axis_index("c")
    t = lax.axis_index("s")
    sems = (sem_a, sem_b)
    sems_si = (sem_si_a, sem_si_b)
    sems_so = (sem_so_a, sem_so_b)
    planes = (plane_a, plane_b)

    @pl.loop(0, _ZB // 16)
    def _zfill(i):
        zbig[pl.ds(i * 16, 16)] = jnp.zeros((16,), jnp.float32)

    for h in range(2):
        plane_h = planes[h]

        @pl.loop(0, _SLAB // _ZB)
        def _zslab(i):
            pltpu.sync_copy(zbig, plane_h.at[pl.ds(t * _SLAB + i * _ZB, _ZB)])

    plsc.subcore_barrier()

    @pl.loop(0, _BPC // 2)
    def _pair(g):
        for h in range(2):
            b = c * _BPC + g * 2 + h
            sem_h = sems[h]
            plane_h = planes[h]
            idx_h = idx_v.at[h]

            @pl.when(g > 0)
            def _recycle():
                pltpu.make_async_copy(spat_hbm.at[pl.ds(0, _SPT)],
                                      sp_v.at[h], sems_so[h]).wait()
                pltpu.make_async_copy(
                    out_hbm.at[pl.ds(0, _SLAB)],
                    plane_h.at[pl.ds(t * _SLAB, _SLAB)], sem_h).wait()
                plsc.subcore_barrier()
                for j in range(_NSTR):
                    pltpu.sync_copy(zbig.at[pl.ds(0, 128)],
                                    plane_h.at[idx_h.at[j]])
                plsc.subcore_barrier()

            pltpu.async_copy(
                spat_hbm.at[pl.ds(b * _SPAT + t * _SPT, _SPT)],
                sp_v.at[h], sems_si[h])

            pltpu.sync_copy(idx_hbm.at[b * _NT + t], idx_h)
            pltpu.sync_copy(data_hbm.at[b * _NT + t], data_v)
            for j in range(_NSTR):
                pltpu.sync_copy(data_v.at[j], plane_h.at[idx_h.at[j]])
            plsc.subcore_barrier()
            pltpu.async_copy(
                plane_h.at[pl.ds(t * _SLAB, _SLAB)],
                out_hbm.at[pl.ds((b * _CO + _C) * _HW + t * _SLAB, _SLAB)],
                sem_h)
            pltpu.make_async_copy(spat_hbm.at[pl.ds(0, _SPT)],
                                  sp_v.at[h], sems_si[h]).wait()
            pltpu.async_copy(
                sp_v.at[h],
                out_hbm.at[pl.ds(b * _CO * _HW + t * _SPT, _SPT)],
                sems_so[h])

    for h in range(2):
        pltpu.make_async_copy(spat_hbm.at[pl.ds(0, _SPT)],
                              sp_v.at[h], sems_so[h]).wait()
        pltpu.make_async_copy(out_hbm.at[pl.ds(0, _SLAB)],
                              planes[h].at[pl.ds(t * _SLAB, _SLAB)],
                              sems[h]).wait()


def _make_sc_scatter():
    return pl.kernel(
        _sc_body,
        out_type=(),
        mesh=plsc.VectorSubcoreMesh(core_axis_name="c", subcore_axis_name="s"),
        scratch_types=[
            pltpu.VMEM_SHARED((_PLANE,), jnp.float32),
            pltpu.VMEM_SHARED((_PLANE,), jnp.float32),
            pltpu.VMEM((2, _NSTR, 128), jnp.int32),
            pltpu.VMEM((_NSTR, 128), jnp.float32),
            pltpu.VMEM((2, _SPT), jnp.float32),
            pltpu.VMEM((_ZB,), jnp.float32),
            pltpu.SemaphoreType.DMA,
            pltpu.SemaphoreType.DMA,
            pltpu.SemaphoreType.DMA,
            pltpu.SemaphoreType.DMA,
            pltpu.SemaphoreType.DMA,
            pltpu.SemaphoreType.DMA,
        ],
    )


def kernel(spatial_info, entity_embeddings, locations, W_proj, b_proj):
    lh = locations[..., 0].reshape(_B, 1, _N)
    lw = locations[..., 1].reshape(_B, 1, _N)
    b3 = b_proj.reshape(1, 1, _D)
    out0, data, idxw = pl.pallas_call(
        _tc_body,
        grid=(_B,),
        in_specs=[
            pl.BlockSpec((1, _N, _DIN), lambda b: (b, 0, 0)),
            pl.BlockSpec((1, 1, _N), lambda b: (b, 0, 0)),
            pl.BlockSpec((1, 1, _N), lambda b: (b, 0, 0)),
            pl.BlockSpec((_DIN, _D), lambda b: (0, 0)),
            pl.BlockSpec((1, 1, _D), lambda b: (0, 0, 0)),
        ],
        out_specs=[
            pl.BlockSpec((1, 1, 8, _W), lambda b: (0, 0, 0, 0)),
            pl.BlockSpec((_NT, 8, 128), lambda b: (b, 0, 0)),
            pl.BlockSpec((_NT, 8, 128), lambda b: (b, 0, 0)),
        ],
        out_shape=[
            jax.ShapeDtypeStruct((_B, _CO, _H, _W), jnp.float32),
            jax.ShapeDtypeStruct((_B * _NT, 8, 128), jnp.float32),
            jax.ShapeDtypeStruct((_B * _NT, 8, 128), jnp.int32),
        ],
    )(entity_embeddings, lh, lw, W_proj, b3)
    data_t = data
    idx_t = idxw
    spat_flat = spatial_info.reshape(_B * _SPAT)
    out_ref = jax.new_ref(out0.reshape(_TOTAL))
    _make_sc_scatter()(data_t, idx_t, spat_flat, out_ref)
    return jax.freeze(out_ref).reshape(_B, _CO, _H, _W)


# speedup vs baseline: 9.5653x; 1.1836x over previous
import jax
import jax.numpy as jnp
from jax import lax
from jax.experimental import pallas as pl
from jax.experimental.pallas import tpu as pltpu
from jax.experimental.pallas import tpu_sc as plsc

_B, _C, _H, _W = 16, 20, 128, 128
_N, _DIN, _D = 512, 256, 32
_HW = _H * _W
_CO = _C + _D
_TOTAL = _B * _CO * _HW
_SPAT = _C * _HW
_PLANE = _D * _HW
_NT = 16
_SLAB = _PLANE // _NT
_SPT = _SPAT // _NT
_WPB = _N * _D
_WPT = _WPB // _NT
_NSTR = _WPT // 128
_ZB = 2048
_BPC = _B // 2


def _tc_body(emb_ref, lh_ref, lw_ref, w_ref, b_ref,
             out_ref, data_ref, idx_ref):
    out_ref[...] = jnp.zeros((1, 1, 8, _W), jnp.float32)
    proj = jnp.dot(emb_ref[0], w_ref[...], preferred_element_type=jnp.float32)
    proj = jnp.maximum(proj + b_ref[0, 0][None, :], 0.0)
    lh = jnp.clip(lh_ref[0, 0], 0, _H - 1)
    lw = jnp.clip(lw_ref[0, 0], 0, _W - 1)
    p = lh * _W + lw
    same = p[:, None] == p[None, :]
    col = lax.broadcasted_iota(jnp.int32, (_N, _N), 1)
    row = lax.broadcasted_iota(jnp.int32, (_N, _N), 0)
    has_later = jnp.any(same & (col > row), axis=1)
    sel = jnp.where(same & ~has_later[None, :], 1.0, 0.0)
    dataT = lax.dot_general(proj, sel, (((0,), (1,)), ((), ())),
                            preferred_element_type=jnp.float32)
    data_ref[...] = dataT.reshape(_NT, 8, 128)
    p8 = jnp.concatenate([p.reshape(4, 128), p.reshape(4, 128)], axis=0)
    dfac = 2 * lax.broadcasted_iota(jnp.int32, (_NT, 8, 128), 0) + \
        lax.broadcasted_iota(jnp.int32, (_NT, 8, 128), 1) // 4
    idx_ref[...] = p8[None] + dfac * _HW


def _sc_body(data_hbm, idx_hbm, spat_hbm, out_hbm,
             plane_a, plane_b, idx_v, data_v, sp_v, zbig,
             sem_a, sem_b, sem_si_a, sem_si_b, sem_so_a, sem_so_b,
             sem_ld, sem_sc):
    c = lax.axis_index("c")
    t = lax.axis_index("s")
    sems = (sem_a, sem_b)
    sems_si = (sem_si_a, sem_si_b)
    sems_so = (sem_so_a, sem_so_b)
    planes = (plane_a, plane_b)

    @pl.loop(0, _ZB // 16)
    def _zfill(i):
        zbig[pl.ds(i * 16, 16)] = jnp.zeros((16,), jnp.float32)

    for h in range(2):
        plane_h = planes[h]

        @pl.loop(0, _SLAB // _ZB)
        def _zslab(i):
            pltpu.sync_copy(zbig, plane_h.at[pl.ds(t * _SLAB + i * _ZB, _ZB)])

    plsc.subcore_barrier()

    @pl.loop(0, _BPC // 2)
    def _pair(g):
        for h in range(2):
            b = c * _BPC + g * 2 + h
            sem_h = sems[h]
            plane_h = planes[h]
            cur = idx_v.at[(g % 2) * 2 + h]
            old = idx_v.at[(1 - g % 2) * 2 + h]

            pltpu.async_copy(idx_hbm.at[b * _NT + t], cur, sem_ld)
            pltpu.async_copy(data_hbm.at[b * _NT + t], data_v, sem_ld)

            @pl.when(g > 0)
            def _recycle():
                pltpu.make_async_copy(spat_hbm.at[pl.ds(0, _SPT)],
                                      sp_v.at[h], sems_so[h]).wait()
                pltpu.make_async_copy(
                    out_hbm.at[pl.ds(0, _SLAB)],
                    plane_h.at[pl.ds(t * _SLAB, _SLAB)], sem_h).wait()
                plsc.subcore_barrier()
                for j in range(_NSTR):
                    pltpu.async_copy(zbig.at[pl.ds(0, 128)],
                                     plane_h.at[old.at[j]], sem_sc)
                pltpu.make_async_copy(data_hbm.at[0],
                                      data_v, sem_sc).wait()
                plsc.subcore_barrier()

            pltpu.async_copy(
                spat_hbm.at[pl.ds(b * _SPAT + t * _SPT, _SPT)],
                sp_v.at[h], sems_si[h])

            pltpu.make_async_copy(idx_hbm.at[0], cur, sem_ld).wait()
            pltpu.make_async_copy(data_hbm.at[0],
                                  data_v, sem_ld).wait()
            for j in range(_NSTR):
                pltpu.async_copy(data_v.at[j], plane_h.at[cur.at[j]], sem_sc)
            pltpu.make_async_copy(data_hbm.at[0],
                                  data_v, sem_sc).wait()
            plsc.subcore_barrier()
            pltpu.async_copy(
                plane_h.at[pl.ds(t * _SLAB, _SLAB)],
                out_hbm.at[pl.ds((b * _CO + _C) * _HW + t * _SLAB, _SLAB)],
                sem_h)
            pltpu.make_async_copy(spat_hbm.at[pl.ds(0, _SPT)],
                                  sp_v.at[h], sems_si[h]).wait()
            pltpu.async_copy(
                sp_v.at[h],
                out_hbm.at[pl.ds(b * _CO * _HW + t * _SPT, _SPT)],
                sems_so[h])

    for h in range(2):
        pltpu.make_async_copy(spat_hbm.at[pl.ds(0, _SPT)],
                              sp_v.at[h], sems_so[h]).wait()
        pltpu.make_async_copy(out_hbm.at[pl.ds(0, _SLAB)],
                              planes[h].at[pl.ds(t * _SLAB, _SLAB)],
                              sems[h]).wait()


def _make_sc_scatter():
    return pl.kernel(
        _sc_body,
        out_type=(),
        mesh=plsc.VectorSubcoreMesh(core_axis_name="c", subcore_axis_name="s"),
        scratch_types=[
            pltpu.VMEM_SHARED((_PLANE,), jnp.float32),
            pltpu.VMEM_SHARED((_PLANE,), jnp.float32),
            pltpu.VMEM((4, _NSTR, 128), jnp.int32),
            pltpu.VMEM((_NSTR, 128), jnp.float32),
            pltpu.VMEM((2, _SPT), jnp.float32),
            pltpu.VMEM((_ZB,), jnp.float32),
            pltpu.SemaphoreType.DMA,
            pltpu.SemaphoreType.DMA,
            pltpu.SemaphoreType.DMA,
            pltpu.SemaphoreType.DMA,
            pltpu.SemaphoreType.DMA,
            pltpu.SemaphoreType.DMA,
            pltpu.SemaphoreType.DMA,
            pltpu.SemaphoreType.DMA,
        ],
    )


def kernel(spatial_info, entity_embeddings, locations, W_proj, b_proj):
    lh = locations[..., 0].reshape(_B, 1, _N)
    lw = locations[..., 1].reshape(_B, 1, _N)
    b3 = b_proj.reshape(1, 1, _D)
    out0, data, idxw = pl.pallas_call(
        _tc_body,
        grid=(_B,),
        in_specs=[
            pl.BlockSpec((1, _N, _DIN), lambda b: (b, 0, 0)),
            pl.BlockSpec((1, 1, _N), lambda b: (b, 0, 0)),
            pl.BlockSpec((1, 1, _N), lambda b: (b, 0, 0)),
            pl.BlockSpec((_DIN, _D), lambda b: (0, 0)),
            pl.BlockSpec((1, 1, _D), lambda b: (0, 0, 0)),
        ],
        out_specs=[
            pl.BlockSpec((1, 1, 8, _W), lambda b: (0, 0, 0, 0)),
            pl.BlockSpec((_NT, 8, 128), lambda b: (b, 0, 0)),
            pl.BlockSpec((_NT, 8, 128), lambda b: (b, 0, 0)),
        ],
        out_shape=[
            jax.ShapeDtypeStruct((_B, _CO, _H, _W), jnp.float32),
            jax.ShapeDtypeStruct((_B * _NT, 8, 128), jnp.float32),
            jax.ShapeDtypeStruct((_B * _NT, 8, 128), jnp.int32),
        ],
    )(entity_embeddings, lh, lw, W_proj, b3)
    data_t = data
    idx_t = idxw
    spat_flat = spatial_info.reshape(_B * _SPAT)
    out_ref = jax.new_ref(out0.reshape(_TOTAL))
    _make_sc_scatter()(data_t, idx_t, spat_flat, out_ref)
    return jax.freeze(out_ref).reshape(_B, _CO, _H, _W)


# speedup vs baseline: 9.9669x; 1.0420x over previous
import jax
import jax.numpy as jnp
from jax import lax
from jax.experimental import pallas as pl
from jax.experimental.pallas import tpu as pltpu
from jax.experimental.pallas import tpu_sc as plsc

_B, _C, _H, _W = 16, 20, 128, 128
_N, _DIN, _D = 512, 256, 32
_HW = _H * _W
_CO = _C + _D
_TOTAL = _B * _CO * _HW
_SPAT = _C * _HW
_PLANE = _D * _HW
_NT = 16
_SLAB = _PLANE // _NT
_SPT = _SPAT // _NT
_WPB = _N * _D
_WPT = _WPB // _NT
_NSTR = _WPT // 128
_ZB = 2048
_BPC = _B // 2


def _tc_body(emb_ref, lh_ref, lw_ref, w_ref, b_ref,
             out_ref, data_ref, idx_ref):
    out_ref[...] = jnp.zeros((1, 1, 8, _W), jnp.float32)
    projT = lax.dot_general(w_ref[...], emb_ref[0], (((0,), (1,)), ((), ())),
                            preferred_element_type=jnp.float32)
    projT = jnp.maximum(projT + b_ref[0, 0][:, None], 0.0)
    lh = jnp.clip(lh_ref[0, 0], 0, _H - 1)
    lw = jnp.clip(lw_ref[0, 0], 0, _W - 1)
    p = lh * _W + lw
    same = p[:, None] == p[None, :]
    col = lax.broadcasted_iota(jnp.int32, (_N, _N), 1)
    row = lax.broadcasted_iota(jnp.int32, (_N, _N), 0)
    has_later = jnp.any(same & (col > row), axis=1)
    data_ref[...] = projT.reshape(_NT, 8, 128)
    p8 = jnp.concatenate([p.reshape(4, 128), p.reshape(4, 128)], axis=0)
    hl = has_later.astype(jnp.int32)
    hl8 = jnp.concatenate([hl.reshape(4, 128), hl.reshape(4, 128)], axis=0)
    dfac = 2 * lax.broadcasted_iota(jnp.int32, (_NT, 8, 128), 0) + \
        lax.broadcasted_iota(jnp.int32, (_NT, 8, 128), 1) // 4
    idx_ref[...] = jnp.where(hl8[None] > 0, _PLANE + dfac,
                             p8[None] + dfac * _HW)


def _sc_body(data_hbm, idx_hbm, spat_hbm, out_hbm,
             plane_a, plane_b, idx_v, data_v, sp_v, zbig,
             sem_a, sem_b, sem_si_a, sem_si_b, sem_so_a, sem_so_b,
             sem_ld, sem_sc):
    c = lax.axis_index("c")
    t = lax.axis_index("s")
    sems = (sem_a, sem_b)
    sems_si = (sem_si_a, sem_si_b)
    sems_so = (sem_so_a, sem_so_b)
    planes = (plane_a, plane_b)

    @pl.loop(0, _ZB // 16)
    def _zfill(i):
        zbig[pl.ds(i * 16, 16)] = jnp.zeros((16,), jnp.float32)

    for h in range(2):
        plane_h = planes[h]

        @pl.loop(0, _SLAB // _ZB)
        def _zslab(i):
            pltpu.sync_copy(zbig, plane_h.at[pl.ds(t * _SLAB + i * _ZB, _ZB)])

    plsc.subcore_barrier()

    @pl.loop(0, _BPC // 2)
    def _pair(g):
        for h in range(2):
            b = c * _BPC + g * 2 + h
            sem_h = sems[h]
            plane_h = planes[h]
            cur = idx_v.at[(g % 2) * 2 + h]
            old = idx_v.at[(1 - g % 2) * 2 + h]

            pltpu.async_copy(idx_hbm.at[b * _NT + t], cur, sem_ld)
            pltpu.async_copy(data_hbm.at[b * _NT + t], data_v, sem_ld)

            @pl.when(g > 0)
            def _recycle():
                pltpu.make_async_copy(spat_hbm.at[pl.ds(0, _SPT)],
                                      sp_v.at[h], sems_so[h]).wait()
                pltpu.make_async_copy(
                    out_hbm.at[pl.ds(0, _SLAB)],
                    plane_h.at[pl.ds(t * _SLAB, _SLAB)], sem_h).wait()
                plsc.subcore_barrier()
                for j in range(_NSTR):
                    pltpu.async_copy(zbig.at[pl.ds(0, 128)],
                                     plane_h.at[old.at[j]], sem_sc)
                pltpu.make_async_copy(data_hbm.at[0],
                                      data_v, sem_sc).wait()
                plsc.subcore_barrier()

            pltpu.async_copy(
                spat_hbm.at[pl.ds(b * _SPAT + t * _SPT, _SPT)],
                sp_v.at[h], sems_si[h])

            pltpu.make_async_copy(idx_hbm.at[0], cur, sem_ld).wait()
            pltpu.make_async_copy(data_hbm.at[0],
                                  data_v, sem_ld).wait()
            for j in range(_NSTR):
                pltpu.async_copy(data_v.at[j], plane_h.at[cur.at[j]], sem_sc)
            pltpu.make_async_copy(data_hbm.at[0],
                                  data_v, sem_sc).wait()
            plsc.subcore_barrier()
            pltpu.async_copy(
                plane_h.at[pl.ds(t * _SLAB, _SLAB)],
                out_hbm.at[pl.ds((b * _CO + _C) * _HW + t * _SLAB, _SLAB)],
                sem_h)
            pltpu.make_async_copy(spat_hbm.at[pl.ds(0, _SPT)],
                                  sp_v.at[h], sems_si[h]).wait()
            pltpu.async_copy(
                sp_v.at[h],
                out_hbm.at[pl.ds(b * _CO * _HW + t * _SPT, _SPT)],
                sems_so[h])

    for h in range(2):
        pltpu.make_async_copy(spat_hbm.at[pl.ds(0, _SPT)],
                              sp_v.at[h], sems_so[h]).wait()
        pltpu.make_async_copy(out_hbm.at[pl.ds(0, _SLAB)],
                              planes[h].at[pl.ds(t * _SLAB, _SLAB)],
                              sems[h]).wait()


def _make_sc_scatter():
    return pl.kernel(
        _sc_body,
        out_type=(),
        mesh=plsc.VectorSubcoreMesh(core_axis_name="c", subcore_axis_name="s"),
        scratch_types=[
            pltpu.VMEM_SHARED((_PLANE + 128,), jnp.float32),
            pltpu.VMEM_SHARED((_PLANE + 128,), jnp.float32),
            pltpu.VMEM((4, _NSTR, 128), jnp.int32),
            pltpu.VMEM((_NSTR, 128), jnp.float32),
            pltpu.VMEM((2, _SPT), jnp.float32),
            pltpu.VMEM((_ZB,), jnp.float32),
            pltpu.SemaphoreType.DMA,
            pltpu.SemaphoreType.DMA,
            pltpu.SemaphoreType.DMA,
            pltpu.SemaphoreType.DMA,
            pltpu.SemaphoreType.DMA,
            pltpu.SemaphoreType.DMA,
            pltpu.SemaphoreType.DMA,
            pltpu.SemaphoreType.DMA,
        ],
    )


def kernel(spatial_info, entity_embeddings, locations, W_proj, b_proj):
    lh = locations[..., 0].reshape(_B, 1, _N)
    lw = locations[..., 1].reshape(_B, 1, _N)
    b3 = b_proj.reshape(1, 1, _D)
    out0, data, idxw = pl.pallas_call(
        _tc_body,
        grid=(_B,),
        in_specs=[
            pl.BlockSpec((1, _N, _DIN), lambda b: (b, 0, 0)),
            pl.BlockSpec((1, 1, _N), lambda b: (b, 0, 0)),
            pl.BlockSpec((1, 1, _N), lambda b: (b, 0, 0)),
            pl.BlockSpec((_DIN, _D), lambda b: (0, 0)),
            pl.BlockSpec((1, 1, _D), lambda b: (0, 0, 0)),
        ],
        out_specs=[
            pl.BlockSpec((1, 1, 8, _W), lambda b: (0, 0, 0, 0)),
            pl.BlockSpec((_NT, 8, 128), lambda b: (b, 0, 0)),
            pl.BlockSpec((_NT, 8, 128), lambda b: (b, 0, 0)),
        ],
        out_shape=[
            jax.ShapeDtypeStruct((_B, _CO, _H, _W), jnp.float32),
            jax.ShapeDtypeStruct((_B * _NT, 8, 128), jnp.float32),
            jax.ShapeDtypeStruct((_B * _NT, 8, 128), jnp.int32),
        ],
    )(entity_embeddings, lh, lw, W_proj, b3)
    data_t = data
    idx_t = idxw
    spat_flat = spatial_info.reshape(_B * _SPAT)
    out_ref = jax.new_ref(out0.reshape(_TOTAL))
    _make_sc_scatter()(data_t, idx_t, spat_flat, out_ref)
    return jax.freeze(out_ref).reshape(_B, _CO, _H, _W)
